# Initial kernel scaffold; baseline (speedup 1.0000x reference)
import jax
import jax.numpy as jnp
from jax.experimental import pallas as pl


def kernel(x, edge_index, edge_weight, Wr_root, Wr_k, br, Wu_root, Wu_k, bu, Wc_root, Wc_k, bc, Wg, bg, Wd_root, Wd_k, bd, Wo, bo):
    raise NotImplementedError("write your pallas kernel here")


# speedup vs baseline: 4.4390x; 4.4390x over previous
import functools

import jax
import jax.numpy as jnp
from jax import lax
from jax.experimental import pallas as pl
from jax.experimental.pallas import tpu as pltpu
from jax.experimental.pallas import tpu_sc as plsc

_D = 1
_H = 32
_M = 32
_GDIM = (_D + 2) * _M
_RNN_IN = _D + _GDIM + _H
_XH = _RNN_IN + _H

_N = 10000
_E = 320000
_NBLK = 2000
_CH = 8000


def _sc_mesh():
    return plsc.VectorSubcoreMesh(core_axis_name="c", subcore_axis_name="s")


def _prop_body(n_in, w, refs):
    C = w // 32
    yT_refs = refs[:n_in]
    src_hbm, dst_hbm, ew_hbm, out_hbm = refs[n_in:n_in + 4]
    tabs = refs[n_in + 4:n_in + 4 + C]
    accs = refs[n_in + 4 + C:n_in + 4 + 2 * C]
    srcb, dstb, ewb = refs[n_in + 4 + 2 * C:n_in + 7 + 2 * C]
    wid = lax.axis_index("s") * 2 + lax.axis_index("c")

    for j in range(C):
        col = wid * C + j
        pltpu.sync_copy(yT_refs[0].at[col], tabs[j])
        for extra in yT_refs[1:]:
            pltpu.sync_copy(extra.at[col], accs[j])

    def add_extra(i, _):
        sl = pl.ds(i * 16, 16)
        for j in range(C):
            tabs[j][sl] = tabs[j][sl] + accs[j][sl]
        return 0
    if n_in == 2:
        lax.fori_loop(0, _N // 16, add_extra, 0)

    def zero(i, _):
        z = jnp.zeros((16,), jnp.float32)
        for j in range(C):
            accs[j][pl.ds(i * 16, 16)] = z
        return 0
    lax.fori_loop(0, _N // 16, zero, 0)

    def chunk(k, _):
        base = k * _CH
        pltpu.sync_copy(src_hbm.at[pl.ds(base, _CH)], srcb)
        pltpu.sync_copy(dst_hbm.at[pl.ds(base, _CH)], dstb)
        pltpu.sync_copy(ew_hbm.at[pl.ds(base, _CH)], ewb)

        def step(i, _):
            s = srcb[pl.ds(i * 16, 16)]
            d = dstb[pl.ds(i * 16, 16)]
            g = ewb[pl.ds(i * 16, 16)]
            for j in range(C):
                v = plsc.load_gather(tabs[j], [s])
                plsc.addupdate_scatter(accs[j], [d], v * g)
            return 0
        lax.fori_loop(0, _CH // 16, step, 0)
        return 0
    lax.fori_loop(0, _E // _CH, chunk, 0)

    for j in range(C):
        pltpu.sync_copy(accs[j], out_hbm.at[wid * C + j])


@functools.lru_cache(maxsize=None)
def _make_prop(n_in, w):
    C = w // 32
    scratch = (
        [pltpu.VMEM((_N,), jnp.float32) for _ in range(2 * C)] +
        [pltpu.VMEM((_CH,), jnp.int32),
         pltpu.VMEM((_CH,), jnp.int32),
         pltpu.VMEM((_CH,), jnp.float32)]
    )

    @functools.partial(
        pl.kernel,
        mesh=_sc_mesh(),
        out_type=jax.ShapeDtypeStruct((w, _N), jnp.float32),
        scratch_types=scratch,
        compiler_params=pltpu.CompilerParams(needs_layout_passes=False),
    )
    def k(*refs):
        _prop_body(n_in, w, refs)

    return k


def _prop1_32(*a):
    return _make_prop(1, 32)(*a)


def _prop2_32(*a):
    return _make_prop(2, 32)(*a)


def _prop1_64(*a):
    return _make_prop(1, 64)(*a)


def _prop2_64(*a):
    return _make_prop(2, 64)(*a)


def _full(shape):
    nd = len(shape)
    return pl.BlockSpec(shape, lambda i, _nd=nd: (0,) * _nd)


def _rows(feat):
    return pl.BlockSpec((_NBLK, feat), lambda i: (i, 0))


def _gmm(lin):
    mu = lin[:, : _D * _M]
    sigma = jax.nn.softplus(lin[:, _D * _M: _D * _M + _M])
    pi = jax.nn.softmax(lin[:, _D * _M + _M:], axis=-1)
    return jnp.concatenate([mu, sigma, pi], axis=-1)


def _mm(a, b):
    return jax.lax.dot_general(a, b, (((1,), (0,)), ((), ())),
                               preferred_element_type=jnp.float32)


def _tca_body(x_ref, h_ref, wg_ref, bg_ref, wd_ref, bd_ref,
              xh1_ref, zdr_ref, zd1_ref):
    h = h_ref[...]
    g = _mm(h, wg_ref[...]) + bg_ref[...]
    xh1 = _gmm(g)
    xh1_ref[...] = xh1
    dec_in = jnp.concatenate([x_ref[...], xh1, h], axis=-1)
    zd = _mm(dec_in, wd_ref[...])
    zdr_ref[...] = zd[:, :_H] + bd_ref[...]
    zd1_ref[...] = zd[:, _H:]


def _tca(x_s, h, Wg, bg2, Wd_all, bd2):
    return pl.pallas_call(
        _tca_body,
        grid=(_N // _NBLK,),
        in_specs=[_rows(_D), _rows(_H), _full((_H, _GDIM)), _full((1, _GDIM)),
                  _full((_RNN_IN, 2 * _H)), _full((1, _H))],
        out_specs=[_rows(_GDIM), _rows(_H), _rows(_H)],
        out_shape=[jax.ShapeDtypeStruct((_N, _GDIM), jnp.float32),
                   jax.ShapeDtypeStruct((_N, _H), jnp.float32),
                   jax.ShapeDtypeStruct((_N, _H), jnp.float32)],
    )(x_s, h, Wg, bg2, Wd_all, bd2)


def _tcb_body(x_ref, h_ref, zdr_ref, p1_ref, wo_ref, bo_ref, wru_ref, bru_ref,
              rep_ref, xh2_ref, zrur_ref, zru1_ref, zru2_ref):
    rep = jnp.maximum(zdr_ref[...] + p1_ref[...], 0.0)
    rep_ref[...] = rep
    o = _mm(rep, wo_ref[...]) + bo_ref[...]
    xh2 = jnp.concatenate([x_ref[...], _gmm(o), rep], axis=-1)
    xh2_ref[...] = xh2
    xh = jnp.concatenate([xh2, h_ref[...]], axis=-1)
    zru = _mm(xh, wru_ref[...])
    zrur_ref[...] = zru[:, : 2 * _H] + bru_ref[...]
    zru1_ref[...] = zru[:, 2 * _H: 4 * _H]
    zru2_ref[...] = zru[:, 4 * _H:]


def _tcb(x_s, h, zdr, p1, Wo, bo2, Wru_all, bru2):
    return pl.pallas_call(
        _tcb_body,
        grid=(_N // _NBLK,),
        in_specs=[_rows(_D), _rows(_H), _rows(_H), _rows(_H),
                  _full((_H, _GDIM)), _full((1, _GDIM)),
                  _full((_XH, 6 * _H)), _full((1, 2 * _H))],
        out_specs=[_rows(_H), _rows(_RNN_IN), _rows(2 * _H),
                   _rows(2 * _H), _rows(2 * _H)],
        out_shape=[jax.ShapeDtypeStruct((_N, _H), jnp.float32),
                   jax.ShapeDtypeStruct((_N, _RNN_IN), jnp.float32),
                   jax.ShapeDtypeStruct((_N, 2 * _H), jnp.float32),
                   jax.ShapeDtypeStruct((_N, 2 * _H), jnp.float32),
                   jax.ShapeDtypeStruct((_N, 2 * _H), jnp.float32)],
    )(x_s, h, zdr, p1, Wo, bo2, Wru_all, bru2)


def _tcc_body(xh2_ref, h_ref, zrur_ref, t3_ref, wc_ref, bc_ref,
              u_ref, zcr_ref, zc1_ref, zc2_ref):
    ru = jax.nn.sigmoid(zrur_ref[...] + t3_ref[...])
    r = ru[:, :_H]
    u_ref[...] = ru[:, _H:]
    xrh = jnp.concatenate([xh2_ref[...], r * h_ref[...]], axis=-1)
    zc = _mm(xrh, wc_ref[...])
    zcr_ref[...] = zc[:, :_H] + bc_ref[...]
    zc1_ref[...] = zc[:, _H: 2 * _H]
    zc2_ref[...] = zc[:, 2 * _H:]


def _tcc(xh2, h, zrur, t3, Wc_all, bc2):
    return pl.pallas_call(
        _tcc_body,
        grid=(_N // _NBLK,),
        in_specs=[_rows(_RNN_IN), _rows(_H), _rows(2 * _H), _rows(2 * _H),
                  _full((_XH, 3 * _H)), _full((1, _H))],
        out_specs=[_rows(_H), _rows(_H), _rows(_H), _rows(_H)],
        out_shape=[jax.ShapeDtypeStruct((_N, _H), jnp.float32)] * 4,
    )(xh2, h, zrur, t3, Wc_all, bc2)


def _tcd_body(zcr_ref, t5_ref, u_ref, h_ref, hn_ref):
    c = jnp.tanh(zcr_ref[...] + t5_ref[...])
    u = u_ref[...]
    hn_ref[...] = u * h_ref[...] + (1.0 - u) * c


def _tcd(zcr, t5, u, h):
    return pl.pallas_call(
        _tcd_body,
        grid=(_N // _NBLK,),
        in_specs=[_rows(_H)] * 4,
        out_specs=_rows(_H),
        out_shape=jax.ShapeDtypeStruct((_N, _H), jnp.float32),
    )(zcr, t5, u, h)


def kernel(x, edge_index, edge_weight, Wr_root, Wr_k, br, Wu_root, Wu_k, bu,
           Wc_root, Wc_k, bc, Wg, bg, Wd_root, Wd_k, bd, Wo, bo):
    src = edge_index[0]
    dst = edge_index[1]
    ew = edge_weight

    Wd_all = jnp.concatenate([Wd_root, Wd_k[0]], axis=1)
    Wru_all = jnp.concatenate(
        [Wr_root, Wu_root, Wr_k[0], Wu_k[0], Wr_k[1], Wu_k[1]], axis=1)
    Wc_all = jnp.concatenate([Wc_root, Wc_k[0], Wc_k[1]], axis=1)
    bru2 = jnp.concatenate([br, bu])[None, :]
    bg2 = bg[None, :]
    bd2 = bd[None, :]
    bo2 = bo[None, :]
    bc2 = bc[None, :]

    T = x.shape[1]
    h = jnp.zeros((_N, _H), jnp.float32)
    gens, preds, reprs, states = [], [], [], []
    for t in range(T):
        x_s = x[0, t]
        xh1, zdr, zd1 = _tca(x_s, h, Wg, bg2, Wd_all, bd2)
        p1 = _prop1_32(zd1.T, src, dst, ew).T
        rep, xh2, zrur, zru1, zru2 = _tcb(x_s, h, zdr, p1, Wo, bo2,
                                          Wru_all, bru2)
        t2 = _prop1_64(zru2.T, src, dst, ew)
        t3 = _prop2_64(zru1.T, t2, src, dst, ew).T
        u, zcr, zc1, zc2 = _tcc(xh2, h, zrur, t3, Wc_all, bc2)
        t4 = _prop1_32(zc2.T, src, dst, ew)
        t5 = _prop2_32(zc1.T, t4, src, dst, ew).T
        h = _tcd(zcr, t5, u, h)
        gens.append(xh2)
        preds.append(xh1)
        reprs.append(rep)
        states.append(h)

    generations = jnp.stack(gens, axis=0)[None]
    predictions = jnp.stack(preds, axis=0)[None]
    representations = jnp.stack(reprs, axis=0)[None]
    states_arr = jnp.stack(states, axis=0)[None, None]
    return (generations, predictions, representations, states_arr)


# speedup vs baseline: 6.2899x; 1.4170x over previous
import functools

import jax
import jax.numpy as jnp
from jax import lax
from jax.experimental import pallas as pl
from jax.experimental.pallas import tpu as pltpu
from jax.experimental.pallas import tpu_sc as plsc

_D = 1
_H = 32
_M = 32
_GDIM = (_D + 2) * _M
_RNN_IN = _D + _GDIM + _H
_XH = _RNN_IN + _H

_N = 10000
_E = 320000
_NBLK = 2000
_CH = 4000


def _sc_mesh():
    return plsc.VectorSubcoreMesh(core_axis_name="c", subcore_axis_name="s")


def _prop_body(n_src, w, refs):
    C = w // 16
    srcs = refs[:n_src]
    combo_hbm, ew_hbm, out_hbm = refs[n_src:n_src + 3]
    tabs = refs[n_src + 3:n_src + 3 + C]
    accs = refs[n_src + 3 + C:n_src + 3 + 2 * C]
    cbufs = refs[n_src + 3 + 2 * C:n_src + 5 + 2 * C]
    ebufs = refs[n_src + 5 + 2 * C:n_src + 7 + 2 * C]
    sem_c = refs[n_src + 7 + 2 * C:n_src + 9 + 2 * C]
    sem_e = refs[n_src + 9 + 2 * C:n_src + 11 + 2 * C]
    cid = lax.axis_index("c")
    sid = lax.axis_index("s")
    e2 = _E // 2
    nch = e2 // _CH
    base_sc = cid * e2

    for j in range(C):
        col = sid * C + j
        pltpu.sync_copy(srcs[0].at[col], tabs[j])
        for extra in srcs[1:]:
            pltpu.sync_copy(extra.at[col], accs[j])

            def add_extra(i, _):
                sl = pl.ds(i * 16, 16)
                tabs[j][sl] = tabs[j][sl] + accs[j][sl]
                return 0
            lax.fori_loop(0, _N // 16, add_extra, 0)

    def zero(i, _):
        z = jnp.zeros((16,), jnp.float32)
        for j in range(C):
            accs[j][pl.ds(i * 16, 16)] = z
        return 0
    lax.fori_loop(0, _N // 16, zero, 0)

    pltpu.async_copy(combo_hbm.at[pl.ds(base_sc, _CH)], cbufs[0], sem_c[0])
    pltpu.async_copy(ew_hbm.at[pl.ds(base_sc, _CH)], ebufs[0], sem_e[0])

    def outer(k2, _):
        for b in range(2):
            k = k2 * 2 + b
            nk = k + 1

            @pl.when(nk < nch)
            def _start_next():
                off = base_sc + nk * _CH
                pltpu.async_copy(combo_hbm.at[pl.ds(off, _CH)],
                                 cbufs[1 - b], sem_c[1 - b])
                pltpu.async_copy(ew_hbm.at[pl.ds(off, _CH)],
                                 ebufs[1 - b], sem_e[1 - b])

            pltpu.make_async_copy(combo_hbm.at[pl.ds(0, _CH)],
                                  cbufs[b], sem_c[b]).wait()
            pltpu.make_async_copy(ew_hbm.at[pl.ds(0, _CH)],
                                  ebufs[b], sem_e[b]).wait()

            def step(i, _):
                sl = pl.ds(i * 16, 16)
                cw = cbufs[b][sl]
                g = ebufs[b][sl]
                s = cw >> 14
                d = cw & 16383
                for j in range(C):
                    v = plsc.load_gather(tabs[j], [s])
                    plsc.addupdate_scatter(accs[j], [d], v * g)
                return 0
            lax.fori_loop(0, _CH // 16, step, 0)
        return 0
    lax.fori_loop(0, nch // 2, outer, 0)

    for j in range(C):
        pltpu.sync_copy(accs[j], out_hbm.at[cid, sid * C + j])


@functools.lru_cache(maxsize=None)
def _make_prop(n_src, w):
    C = w // 16
    scratch = (
        [pltpu.VMEM((_N,), jnp.float32) for _ in range(2 * C)] +
        [pltpu.VMEM((_CH,), jnp.int32) for _ in range(2)] +
        [pltpu.VMEM((_CH,), jnp.float32) for _ in range(2)] +
        [pltpu.SemaphoreType.DMA for _ in range(4)]
    )

    @functools.partial(
        pl.kernel,
        mesh=_sc_mesh(),
        out_type=jax.ShapeDtypeStruct((2, w, _N), jnp.float32),
        scratch_types=scratch,
        compiler_params=pltpu.CompilerParams(needs_layout_passes=False),
    )
    def k(*refs):
        _prop_body(n_src, w, refs)

    return k


def _prop(w, *a):
    return _make_prop(len(a) - 2, w)(*a)


def _full(shape):
    nd = len(shape)
    return pl.BlockSpec(shape, lambda i, _nd=nd: (0,) * _nd)


def _rows(feat):
    return pl.BlockSpec((_NBLK, feat), lambda i: (i, 0))


def _gmm(lin):
    mu = lin[:, : _D * _M]
    sigma = jax.nn.softplus(lin[:, _D * _M: _D * _M + _M])
    pi = jax.nn.softmax(lin[:, _D * _M + _M:], axis=-1)
    return jnp.concatenate([mu, sigma, pi], axis=-1)


def _mm(a, b):
    return jax.lax.dot_general(a, b, (((1,), (0,)), ((), ())),
                               preferred_element_type=jnp.float32)


def _tca_body(x_ref, h_ref, wg_ref, bg_ref, wd_ref, bd_ref,
              xh1_ref, zdr_ref, zd1_ref):
    h = h_ref[...]
    g = _mm(h, wg_ref[...]) + bg_ref[...]
    xh1 = _gmm(g)
    xh1_ref[...] = xh1
    dec_in = jnp.concatenate([x_ref[...], xh1, h], axis=-1)
    zd = _mm(dec_in, wd_ref[...])
    zdr_ref[...] = zd[:, :_H] + bd_ref[...]
    zd1_ref[...] = zd[:, _H:]


def _tca(x_s, h, Wg, bg2, Wd_all, bd2):
    return pl.pallas_call(
        _tca_body,
        grid=(_N // _NBLK,),
        in_specs=[_rows(_D), _rows(_H), _full((_H, _GDIM)), _full((1, _GDIM)),
                  _full((_RNN_IN, 2 * _H)), _full((1, _H))],
        out_specs=[_rows(_GDIM), _rows(_H), _rows(_H)],
        out_shape=[jax.ShapeDtypeStruct((_N, _GDIM), jnp.float32),
                   jax.ShapeDtypeStruct((_N, _H), jnp.float32),
                   jax.ShapeDtypeStruct((_N, _H), jnp.float32)],
    )(x_s, h, Wg, bg2, Wd_all, bd2)


def _tcb_body(x_ref, h_ref, zdr_ref, p1a_ref, p1b_ref, wo_ref, bo_ref,
              wru_ref, bru_ref,
              rep_ref, xh2_ref, zrur_ref, zru1_ref, zru2_ref):
    rep = jnp.maximum(zdr_ref[...] + p1a_ref[...] + p1b_ref[...], 0.0)
    rep_ref[...] = rep
    o = _mm(rep, wo_ref[...]) + bo_ref[...]
    xh2 = jnp.concatenate([x_ref[...], _gmm(o), rep], axis=-1)
    xh2_ref[...] = xh2
    xh = jnp.concatenate([xh2, h_ref[...]], axis=-1)
    zru = _mm(xh, wru_ref[...])
    zrur_ref[...] = zru[:, : 2 * _H] + bru_ref[...]
    zru1_ref[...] = zru[:, 2 * _H: 4 * _H]
    zru2_ref[...] = zru[:, 4 * _H:]


def _tcb(x_s, h, zdr, p1a, p1b, Wo, bo2, Wru_all, bru2):
    return pl.pallas_call(
        _tcb_body,
        grid=(_N // _NBLK,),
        in_specs=[_rows(_D), _rows(_H), _rows(_H), _rows(_H), _rows(_H),
                  _full((_H, _GDIM)), _full((1, _GDIM)),
                  _full((_XH, 6 * _H)), _full((1, 2 * _H))],
        out_specs=[_rows(_H), _rows(_RNN_IN), _rows(2 * _H),
                   _rows(2 * _H), _rows(2 * _H)],
        out_shape=[jax.ShapeDtypeStruct((_N, _H), jnp.float32),
                   jax.ShapeDtypeStruct((_N, _RNN_IN), jnp.float32),
                   jax.ShapeDtypeStruct((_N, 2 * _H), jnp.float32),
                   jax.ShapeDtypeStruct((_N, 2 * _H), jnp.float32),
                   jax.ShapeDtypeStruct((_N, 2 * _H), jnp.float32)],
    )(x_s, h, zdr, p1a, p1b, Wo, bo2, Wru_all, bru2)


def _tcc_body(xh2_ref, h_ref, zrur_ref, t3a_ref, t3b_ref, wc_ref, bc_ref,
              u_ref, zcr_ref, zc1_ref, zc2_ref):
    ru = jax.nn.sigmoid(zrur_ref[...] + t3a_ref[...] + t3b_ref[...])
    r = ru[:, :_H]
    u_ref[...] = ru[:, _H:]
    xrh = jnp.concatenate([xh2_ref[...], r * h_ref[...]], axis=-1)
    zc = _mm(xrh, wc_ref[...])
    zcr_ref[...] = zc[:, :_H] + bc_ref[...]
    zc1_ref[...] = zc[:, _H: 2 * _H]
    zc2_ref[...] = zc[:, 2 * _H:]


def _tcc(xh2, h, zrur, t3a, t3b, Wc_all, bc2):
    return pl.pallas_call(
        _tcc_body,
        grid=(_N // _NBLK,),
        in_specs=[_rows(_RNN_IN), _rows(_H), _rows(2 * _H), _rows(2 * _H),
                  _rows(2 * _H), _full((_XH, 3 * _H)), _full((1, _H))],
        out_specs=[_rows(_H), _rows(_H), _rows(_H), _rows(_H)],
        out_shape=[jax.ShapeDtypeStruct((_N, _H), jnp.float32)] * 4,
    )(xh2, h, zrur, t3a, t3b, Wc_all, bc2)


def _tcd_body(zcr_ref, t5a_ref, t5b_ref, u_ref, h_ref, hn_ref):
    c = jnp.tanh(zcr_ref[...] + t5a_ref[...] + t5b_ref[...])
    u = u_ref[...]
    hn_ref[...] = u * h_ref[...] + (1.0 - u) * c


def _tcd(zcr, t5a, t5b, u, h):
    return pl.pallas_call(
        _tcd_body,
        grid=(_N // _NBLK,),
        in_specs=[_rows(_H)] * 5,
        out_specs=_rows(_H),
        out_shape=jax.ShapeDtypeStruct((_N, _H), jnp.float32),
    )(zcr, t5a, t5b, u, h)


def kernel(x, edge_index, edge_weight, Wr_root, Wr_k, br, Wu_root, Wu_k, bu,
           Wc_root, Wc_k, bc, Wg, bg, Wd_root, Wd_k, bd, Wo, bo):
    combo = (edge_index[0] << 14) | edge_index[1]
    ew = edge_weight

    Wd_all = jnp.concatenate([Wd_root, Wd_k[0]], axis=1)
    Wru_all = jnp.concatenate(
        [Wr_root, Wu_root, Wr_k[0], Wu_k[0], Wr_k[1], Wu_k[1]], axis=1)
    Wc_all = jnp.concatenate([Wc_root, Wc_k[0], Wc_k[1]], axis=1)
    bru2 = jnp.concatenate([br, bu])[None, :]
    bg2 = bg[None, :]
    bd2 = bd[None, :]
    bo2 = bo[None, :]
    bc2 = bc[None, :]

    T = x.shape[1]
    h = jnp.zeros((_N, _H), jnp.float32)
    gens, preds, reprs, states = [], [], [], []
    for t in range(T):
        x_s = x[0, t]
        xh1, zdr, zd1 = _tca(x_s, h, Wg, bg2, Wd_all, bd2)
        p1 = jnp.transpose(_prop(32, zd1.T, combo, ew), (0, 2, 1))
        rep, xh2, zrur, zru1, zru2 = _tcb(x_s, h, zdr, p1[0], p1[1],
                                          Wo, bo2, Wru_all, bru2)
        t2 = _prop(64, zru2.T, combo, ew)
        t3 = jnp.transpose(_prop(64, zru1.T, t2[0], t2[1], combo, ew),
                           (0, 2, 1))
        u, zcr, zc1, zc2 = _tcc(xh2, h, zrur, t3[0], t3[1], Wc_all, bc2)
        t4 = _prop(32, zc2.T, combo, ew)
        t5 = jnp.transpose(_prop(32, zc1.T, t4[0], t4[1], combo, ew),
                           (0, 2, 1))
        h = _tcd(zcr, t5[0], t5[1], u, h)
        gens.append(xh2)
        preds.append(xh1)
        reprs.append(rep)
        states.append(h)

    generations = jnp.stack(gens, axis=0)[None]
    predictions = jnp.stack(preds, axis=0)[None]
    representations = jnp.stack(reprs, axis=0)[None]
    states_arr = jnp.stack(states, axis=0)[None, None]
    return (generations, predictions, representations, states_arr)


# speedup vs baseline: 15.1514x; 2.4088x over previous
import functools

import jax
import jax.numpy as jnp
from jax import lax
from jax.experimental import pallas as pl
from jax.experimental.pallas import tpu as pltpu
from jax.experimental.pallas import tpu_sc as plsc

_D = 1
_H = 32
_M = 32
_GDIM = (_D + 2) * _M
_RNN_IN = _D + _GDIM + _H
_XH = _RNN_IN + _H

_N = 10000
_E = 320000
_NBLK = 2000
_CH = 4000


def _sc_mesh():
    return plsc.VectorSubcoreMesh(core_axis_name="c", subcore_axis_name="s")


def _prop_body(n_src, w, refs):
    C = w // 16
    srcs = refs[:n_src]
    combo_hbm, ew_hbm, out_hbm = refs[n_src:n_src + 3]
    tabs = refs[n_src + 3:n_src + 3 + C]
    accs = refs[n_src + 3 + C:n_src + 3 + 2 * C]
    cbufs = refs[n_src + 3 + 2 * C:n_src + 5 + 2 * C]
    ebufs = refs[n_src + 5 + 2 * C:n_src + 7 + 2 * C]
    sem_c = refs[n_src + 7 + 2 * C:n_src + 9 + 2 * C]
    sem_e = refs[n_src + 9 + 2 * C:n_src + 11 + 2 * C]
    cid = lax.axis_index("c")
    sid = lax.axis_index("s")
    e2 = _E // 2
    nch = e2 // _CH
    base_sc = cid * e2

    for j in range(C):
        col = sid * C + j
        pltpu.sync_copy(srcs[0].at[col], tabs[j])
        for extra in srcs[1:]:
            pltpu.sync_copy(extra.at[col], accs[j])

            @plsc.parallel_loop(0, _N // 16, unroll=4)
            def add_extra(i, _j=j):
                sl = pl.ds(i * 16, 16)
                tabs[_j][sl] = tabs[_j][sl] + accs[_j][sl]

    @plsc.parallel_loop(0, _N // 16, unroll=4)
    def zero(i):
        z = jnp.zeros((16,), jnp.float32)
        for j in range(C):
            accs[j][pl.ds(i * 16, 16)] = z

    pltpu.async_copy(combo_hbm.at[pl.ds(base_sc, _CH)], cbufs[0], sem_c[0])
    pltpu.async_copy(ew_hbm.at[pl.ds(base_sc, _CH)], ebufs[0], sem_e[0])

    def outer(k2, _):
        for b in range(2):
            k = k2 * 2 + b
            nk = k + 1

            @pl.when(nk < nch)
            def _start_next():
                off = base_sc + nk * _CH
                pltpu.async_copy(combo_hbm.at[pl.ds(off, _CH)],
                                 cbufs[1 - b], sem_c[1 - b])
                pltpu.async_copy(ew_hbm.at[pl.ds(off, _CH)],
                                 ebufs[1 - b], sem_e[1 - b])

            pltpu.make_async_copy(combo_hbm.at[pl.ds(0, _CH)],
                                  cbufs[b], sem_c[b]).wait()
            pltpu.make_async_copy(ew_hbm.at[pl.ds(0, _CH)],
                                  ebufs[b], sem_e[b]).wait()

            @plsc.parallel_loop(0, _CH // 16, unroll=8)
            def step(i, _b=b):
                sl = pl.ds(i * 16, 16)
                cw = cbufs[_b][sl]
                g = ebufs[_b][sl]
                s = cw >> 14
                d = cw & 16383
                for j in range(C):
                    v = plsc.load_gather(tabs[j], [s])
                    plsc.addupdate_scatter(accs[j], [d], v * g)
        return 0
    lax.fori_loop(0, nch // 2, outer, 0)

    for j in range(C):
        pltpu.sync_copy(accs[j], out_hbm.at[cid, sid * C + j])


@functools.lru_cache(maxsize=None)
def _make_prop(n_src, w):
    C = w // 16
    scratch = (
        [pltpu.VMEM((_N,), jnp.float32) for _ in range(2 * C)] +
        [pltpu.VMEM((_CH,), jnp.int32) for _ in range(2)] +
        [pltpu.VMEM((_CH,), jnp.float32) for _ in range(2)] +
        [pltpu.SemaphoreType.DMA for _ in range(4)]
    )

    @functools.partial(
        pl.kernel,
        mesh=_sc_mesh(),
        out_type=jax.ShapeDtypeStruct((2, w, _N), jnp.float32),
        scratch_types=scratch,
        compiler_params=pltpu.CompilerParams(needs_layout_passes=False),
    )
    def k(*refs):
        _prop_body(n_src, w, refs)

    return k


def _prop(w, *a):
    return _make_prop(len(a) - 2, w)(*a)


def _full(shape):
    nd = len(shape)
    return pl.BlockSpec(shape, lambda i, _nd=nd: (0,) * _nd)


def _rows(feat):
    return pl.BlockSpec((_NBLK, feat), lambda i: (i, 0))


def _gmm(lin):
    mu = lin[:, : _D * _M]
    sigma = jax.nn.softplus(lin[:, _D * _M: _D * _M + _M])
    pi = jax.nn.softmax(lin[:, _D * _M + _M:], axis=-1)
    return jnp.concatenate([mu, sigma, pi], axis=-1)


def _mm(a, b):
    return jax.lax.dot_general(a, b, (((1,), (0,)), ((), ())),
                               preferred_element_type=jnp.float32)


def _tca_body(x_ref, h_ref, wg_ref, bg_ref, wd_ref, bd_ref,
              xh1_ref, zdr_ref, zd1_ref):
    h = h_ref[...]
    g = _mm(h, wg_ref[...]) + bg_ref[...]
    xh1 = _gmm(g)
    xh1_ref[...] = xh1
    dec_in = jnp.concatenate([x_ref[...], xh1, h], axis=-1)
    zd = _mm(dec_in, wd_ref[...])
    zdr_ref[...] = zd[:, :_H] + bd_ref[...]
    zd1_ref[...] = zd[:, _H:]


def _tca(x_s, h, Wg, bg2, Wd_all, bd2):
    return pl.pallas_call(
        _tca_body,
        grid=(_N // _NBLK,),
        in_specs=[_rows(_D), _rows(_H), _full((_H, _GDIM)), _full((1, _GDIM)),
                  _full((_RNN_IN, 2 * _H)), _full((1, _H))],
        out_specs=[_rows(_GDIM), _rows(_H), _rows(_H)],
        out_shape=[jax.ShapeDtypeStruct((_N, _GDIM), jnp.float32),
                   jax.ShapeDtypeStruct((_N, _H), jnp.float32),
                   jax.ShapeDtypeStruct((_N, _H), jnp.float32)],
    )(x_s, h, Wg, bg2, Wd_all, bd2)


def _tcb_body(x_ref, h_ref, zdr_ref, p1a_ref, p1b_ref, wo_ref, bo_ref,
              wru_ref, bru_ref,
              rep_ref, xh2_ref, zrur_ref, zru1_ref, zru2_ref):
    rep = jnp.maximum(zdr_ref[...] + p1a_ref[...] + p1b_ref[...], 0.0)
    rep_ref[...] = rep
    o = _mm(rep, wo_ref[...]) + bo_ref[...]
    xh2 = jnp.concatenate([x_ref[...], _gmm(o), rep], axis=-1)
    xh2_ref[...] = xh2
    xh = jnp.concatenate([xh2, h_ref[...]], axis=-1)
    zru = _mm(xh, wru_ref[...])
    zrur_ref[...] = zru[:, : 2 * _H] + bru_ref[...]
    zru1_ref[...] = zru[:, 2 * _H: 4 * _H]
    zru2_ref[...] = zru[:, 4 * _H:]


def _tcb(x_s, h, zdr, p1a, p1b, Wo, bo2, Wru_all, bru2):
    return pl.pallas_call(
        _tcb_body,
        grid=(_N // _NBLK,),
        in_specs=[_rows(_D), _rows(_H), _rows(_H), _rows(_H), _rows(_H),
                  _full((_H, _GDIM)), _full((1, _GDIM)),
                  _full((_XH, 6 * _H)), _full((1, 2 * _H))],
        out_specs=[_rows(_H), _rows(_RNN_IN), _rows(2 * _H),
                   _rows(2 * _H), _rows(2 * _H)],
        out_shape=[jax.ShapeDtypeStruct((_N, _H), jnp.float32),
                   jax.ShapeDtypeStruct((_N, _RNN_IN), jnp.float32),
                   jax.ShapeDtypeStruct((_N, 2 * _H), jnp.float32),
                   jax.ShapeDtypeStruct((_N, 2 * _H), jnp.float32),
                   jax.ShapeDtypeStruct((_N, 2 * _H), jnp.float32)],
    )(x_s, h, zdr, p1a, p1b, Wo, bo2, Wru_all, bru2)


def _tcc_body(xh2_ref, h_ref, zrur_ref, t3a_ref, t3b_ref, wc_ref, bc_ref,
              u_ref, zcr_ref, zc1_ref, zc2_ref):
    ru = jax.nn.sigmoid(zrur_ref[...] + t3a_ref[...] + t3b_ref[...])
    r = ru[:, :_H]
    u_ref[...] = ru[:, _H:]
    xrh = jnp.concatenate([xh2_ref[...], r * h_ref[...]], axis=-1)
    zc = _mm(xrh, wc_ref[...])
    zcr_ref[...] = zc[:, :_H] + bc_ref[...]
    zc1_ref[...] = zc[:, _H: 2 * _H]
    zc2_ref[...] = zc[:, 2 * _H:]


def _tcc(xh2, h, zrur, t3a, t3b, Wc_all, bc2):
    return pl.pallas_call(
        _tcc_body,
        grid=(_N // _NBLK,),
        in_specs=[_rows(_RNN_IN), _rows(_H), _rows(2 * _H), _rows(2 * _H),
                  _rows(2 * _H), _full((_XH, 3 * _H)), _full((1, _H))],
        out_specs=[_rows(_H), _rows(_H), _rows(_H), _rows(_H)],
        out_shape=[jax.ShapeDtypeStruct((_N, _H), jnp.float32)] * 4,
    )(xh2, h, zrur, t3a, t3b, Wc_all, bc2)


def _tcd_body(zcr_ref, t5a_ref, t5b_ref, u_ref, h_ref, hn_ref):
    c = jnp.tanh(zcr_ref[...] + t5a_ref[...] + t5b_ref[...])
    u = u_ref[...]
    hn_ref[...] = u * h_ref[...] + (1.0 - u) * c


def _tcd(zcr, t5a, t5b, u, h):
    return pl.pallas_call(
        _tcd_body,
        grid=(_N // _NBLK,),
        in_specs=[_rows(_H)] * 5,
        out_specs=_rows(_H),
        out_shape=jax.ShapeDtypeStruct((_N, _H), jnp.float32),
    )(zcr, t5a, t5b, u, h)


def kernel(x, edge_index, edge_weight, Wr_root, Wr_k, br, Wu_root, Wu_k, bu,
           Wc_root, Wc_k, bc, Wg, bg, Wd_root, Wd_k, bd, Wo, bo):
    combo = (edge_index[0] << 14) | edge_index[1]
    ew = edge_weight

    Wd_all = jnp.concatenate([Wd_root, Wd_k[0]], axis=1)
    Wru_all = jnp.concatenate(
        [Wr_root, Wu_root, Wr_k[0], Wu_k[0], Wr_k[1], Wu_k[1]], axis=1)
    Wc_all = jnp.concatenate([Wc_root, Wc_k[0], Wc_k[1]], axis=1)
    bru2 = jnp.concatenate([br, bu])[None, :]
    bg2 = bg[None, :]
    bd2 = bd[None, :]
    bo2 = bo[None, :]
    bc2 = bc[None, :]

    T = x.shape[1]
    h = jnp.zeros((_N, _H), jnp.float32)
    gens, preds, reprs, states = [], [], [], []
    for t in range(T):
        x_s = x[0, t]
        xh1, zdr, zd1 = _tca(x_s, h, Wg, bg2, Wd_all, bd2)
        p1 = jnp.transpose(_prop(32, zd1.T, combo, ew), (0, 2, 1))
        rep, xh2, zrur, zru1, zru2 = _tcb(x_s, h, zdr, p1[0], p1[1],
                                          Wo, bo2, Wru_all, bru2)
        t2 = _prop(64, zru2.T, combo, ew)
        t3 = jnp.transpose(_prop(64, zru1.T, t2[0], t2[1], combo, ew),
                           (0, 2, 1))
        u, zcr, zc1, zc2 = _tcc(xh2, h, zrur, t3[0], t3[1], Wc_all, bc2)
        t4 = _prop(32, zc2.T, combo, ew)
        t5 = jnp.transpose(_prop(32, zc1.T, t4[0], t4[1], combo, ew),
                           (0, 2, 1))
        h = _tcd(zcr, t5[0], t5[1], u, h)
        gens.append(xh2)
        preds.append(xh1)
        reprs.append(rep)
        states.append(h)

    generations = jnp.stack(gens, axis=0)[None]
    predictions = jnp.stack(preds, axis=0)[None]
    representations = jnp.stack(reprs, axis=0)[None]
    states_arr = jnp.stack(states, axis=0)[None, None]
    return (generations, predictions, representations, states_arr)


# speedup vs baseline: 16.4879x; 1.0882x over previous
import functools

import jax
import jax.numpy as jnp
from jax import lax
from jax.experimental import pallas as pl
from jax.experimental.pallas import tpu as pltpu
from jax.experimental.pallas import tpu_sc as plsc

_D = 1
_H = 32
_M = 32
_GDIM = (_D + 2) * _M
_RNN_IN = _D + _GDIM + _H
_XH = _RNN_IN + _H

_N = 10000
_NP = 10240
_E = 320000
_NBLK = 2048
_NG = _NP // _NBLK
_CH = 4000
_C = 4


def _sc_mesh():
    return plsc.VectorSubcoreMesh(core_axis_name="c", subcore_axis_name="s")


def _prop_body(n_src, w, refs):
    C = _C
    n_grp = w // C
    n_parts = 32 // n_grp
    srcs = refs[:n_src]
    combo_hbm, ew_hbm, out_hbm = refs[n_src:n_src + 3]
    tabs = refs[n_src + 3:n_src + 3 + C]
    accs = refs[n_src + 3 + C:n_src + 3 + 2 * C]
    cbufs = refs[n_src + 3 + 2 * C:n_src + 5 + 2 * C]
    ebufs = refs[n_src + 5 + 2 * C:n_src + 7 + 2 * C]
    sem_c = refs[n_src + 7 + 2 * C:n_src + 9 + 2 * C]
    sem_e = refs[n_src + 9 + 2 * C:n_src + 11 + 2 * C]
    cid = lax.axis_index("c")
    sid = lax.axis_index("s")
    grp = sid % n_grp
    part = cid * (16 // n_grp) + sid // n_grp
    ep = _E // n_parts
    nch = ep // _CH
    base_sc = part * ep

    for j in range(C):
        col = grp * C + j
        first = True
        for s in srcs:
            views = [s.at[col]] if len(s.shape) == 2 else [
                s.at[q, col] for q in range(s.shape[0])]
            for v in views:
                if first:
                    pltpu.sync_copy(v, tabs[j])
                    first = False
                    continue
                pltpu.sync_copy(v, accs[j])

                @plsc.parallel_loop(0, _NP // 16, unroll=4)
                def add_extra(i, _j=j):
                    sl = pl.ds(i * 16, 16)
                    tabs[_j][sl] = tabs[_j][sl] + accs[_j][sl]

    @plsc.parallel_loop(0, _NP // 16, unroll=4)
    def zero(i):
        z = jnp.zeros((16,), jnp.float32)
        for j in range(C):
            accs[j][pl.ds(i * 16, 16)] = z

    pltpu.async_copy(combo_hbm.at[pl.ds(base_sc, _CH)], cbufs[0], sem_c[0])
    pltpu.async_copy(ew_hbm.at[pl.ds(base_sc, _CH)], ebufs[0], sem_e[0])

    def outer(k2, _):
        for b in range(2):
            k = k2 * 2 + b
            nk = k + 1

            @pl.when(nk < nch)
            def _start_next():
                off = base_sc + nk * _CH
                pltpu.async_copy(combo_hbm.at[pl.ds(off, _CH)],
                                 cbufs[1 - b], sem_c[1 - b])
                pltpu.async_copy(ew_hbm.at[pl.ds(off, _CH)],
                                 ebufs[1 - b], sem_e[1 - b])

            pltpu.make_async_copy(combo_hbm.at[pl.ds(0, _CH)],
                                  cbufs[b], sem_c[b]).wait()
            pltpu.make_async_copy(ew_hbm.at[pl.ds(0, _CH)],
                                  ebufs[b], sem_e[b]).wait()

            @plsc.parallel_loop(0, _CH // 16, unroll=8)
            def step(i, _b=b):
                sl = pl.ds(i * 16, 16)
                cw = cbufs[_b][sl]
                g = ebufs[_b][sl]
                s = cw >> 14
                d = cw & 16383
                for j in range(C):
                    v = plsc.load_gather(tabs[j], [s])
                    plsc.addupdate_scatter(accs[j], [d], v * g)
        return 0
    lax.fori_loop(0, nch // 2, outer, 0)

    for j in range(C):
        pltpu.sync_copy(accs[j], out_hbm.at[part, grp * C + j])


@functools.lru_cache(maxsize=None)
def _make_prop(src_shapes, w):
    n_parts = 32 // (w // _C)
    scratch = (
        [pltpu.VMEM((_NP,), jnp.float32) for _ in range(2 * _C)] +
        [pltpu.VMEM((_CH,), jnp.int32) for _ in range(2)] +
        [pltpu.VMEM((_CH,), jnp.float32) for _ in range(2)] +
        [pltpu.SemaphoreType.DMA for _ in range(4)]
    )

    @functools.partial(
        pl.kernel,
        mesh=_sc_mesh(),
        out_type=jax.ShapeDtypeStruct((n_parts, w, _NP), jnp.float32),
        scratch_types=scratch,
        compiler_params=pltpu.CompilerParams(needs_layout_passes=False),
    )
    def k(*refs):
        _prop_body(len(src_shapes), w, refs)

    return k


def _prop(w, *a):
    srcs = a[:-2]
    return _make_prop(tuple(s.shape for s in srcs), w)(*a)


def _full(shape):
    nd = len(shape)
    return pl.BlockSpec(shape, lambda i, _nd=nd: (0,) * _nd)


def _rows(feat):
    return pl.BlockSpec((_NBLK, feat), lambda i: (i, 0))


def _cols(feat):
    return _full((feat, _NP))


def _parts(p, feat):
    return _full((p, feat, _NP))


def _seg():
    return pl.ds(pl.multiple_of(pl.program_id(0) * _NBLK, _NBLK), _NBLK)


def _psum_t(p_ref):
    sl = _seg()
    s = p_ref[0, :, sl]
    for q in range(1, p_ref.shape[0]):
        s = s + p_ref[q, :, sl]
    return s.T


def _gmm(lin):
    mu = lin[:, : _D * _M]
    sigma = jax.nn.softplus(lin[:, _D * _M: _D * _M + _M])
    pi = jax.nn.softmax(lin[:, _D * _M + _M:], axis=-1)
    return jnp.concatenate([mu, sigma, pi], axis=-1)


def _mm(a, b):
    return jax.lax.dot_general(a, b, (((1,), (0,)), ((), ())),
                               preferred_element_type=jnp.float32)


def _tca_body(x_ref, h_ref, wg_ref, bg_ref, wd_ref, bd_ref,
              xh1_ref, zdr_ref, zd1_ref):
    h = h_ref[...]
    g = _mm(h, wg_ref[...]) + bg_ref[...]
    xh1 = _gmm(g)
    xh1_ref[...] = xh1
    dec_in = jnp.concatenate([x_ref[...], xh1, h], axis=-1)
    zd = _mm(dec_in, wd_ref[...])
    zdr_ref[...] = zd[:, :_H] + bd_ref[...]
    zd1_ref[:, _seg()] = zd[:, _H:].T


def _tca(x_s, h, Wg, bg2, Wd_all, bd2):
    return pl.pallas_call(
        _tca_body,
        grid=(_NG,),
        in_specs=[_rows(_D), _rows(_H), _full((_H, _GDIM)), _full((1, _GDIM)),
                  _full((_RNN_IN, 2 * _H)), _full((1, _H))],
        out_specs=[_rows(_GDIM), _rows(_H), _cols(_H)],
        out_shape=[jax.ShapeDtypeStruct((_NP, _GDIM), jnp.float32),
                   jax.ShapeDtypeStruct((_NP, _H), jnp.float32),
                   jax.ShapeDtypeStruct((_H, _NP), jnp.float32)],
    )(x_s, h, Wg, bg2, Wd_all, bd2)


def _tcb_body(x_ref, h_ref, zdr_ref, p1_ref, wo_ref, bo_ref,
              wru_ref, bru_ref,
              rep_ref, xh2_ref, zrur_ref, zru1_ref, zru2_ref):
    rep = jnp.maximum(zdr_ref[...] + _psum_t(p1_ref), 0.0)
    rep_ref[...] = rep
    o = _mm(rep, wo_ref[...]) + bo_ref[...]
    xh2 = jnp.concatenate([x_ref[...], _gmm(o), rep], axis=-1)
    xh2_ref[...] = xh2
    xh = jnp.concatenate([xh2, h_ref[...]], axis=-1)
    zru = _mm(xh, wru_ref[...])
    zrur_ref[...] = zru[:, : 2 * _H] + bru_ref[...]
    sl = _seg()
    zru1_ref[:, sl] = zru[:, 2 * _H: 4 * _H].T
    zru2_ref[:, sl] = zru[:, 4 * _H:].T


def _tcb(x_s, h, zdr, p1, Wo, bo2, Wru_all, bru2):
    return pl.pallas_call(
        _tcb_body,
        grid=(_NG,),
        in_specs=[_rows(_D), _rows(_H), _rows(_H), _parts(4, _H),
                  _full((_H, _GDIM)), _full((1, _GDIM)),
                  _full((_XH, 6 * _H)), _full((1, 2 * _H))],
        out_specs=[_rows(_H), _rows(_RNN_IN), _rows(2 * _H),
                   _cols(2 * _H), _cols(2 * _H)],
        out_shape=[jax.ShapeDtypeStruct((_NP, _H), jnp.float32),
                   jax.ShapeDtypeStruct((_NP, _RNN_IN), jnp.float32),
                   jax.ShapeDtypeStruct((_NP, 2 * _H), jnp.float32),
                   jax.ShapeDtypeStruct((2 * _H, _NP), jnp.float32),
                   jax.ShapeDtypeStruct((2 * _H, _NP), jnp.float32)],
    )(x_s, h, zdr, p1, Wo, bo2, Wru_all, bru2)


def _tcc_body(xh2_ref, h_ref, zrur_ref, t3_ref, wc_ref, bc_ref,
              u_ref, zcr_ref, zc1_ref, zc2_ref):
    ru = jax.nn.sigmoid(zrur_ref[...] + _psum_t(t3_ref))
    r = ru[:, :_H]
    u_ref[...] = ru[:, _H:]
    xrh = jnp.concatenate([xh2_ref[...], r * h_ref[...]], axis=-1)
    zc = _mm(xrh, wc_ref[...])
    zcr_ref[...] = zc[:, :_H] + bc_ref[...]
    sl = _seg()
    zc1_ref[:, sl] = zc[:, _H: 2 * _H].T
    zc2_ref[:, sl] = zc[:, 2 * _H:].T


def _tcc(xh2, h, zrur, t3, Wc_all, bc2):
    return pl.pallas_call(
        _tcc_body,
        grid=(_NG,),
        in_specs=[_rows(_RNN_IN), _rows(_H), _rows(2 * _H),
                  _parts(2, 2 * _H), _full((_XH, 3 * _H)), _full((1, _H))],
        out_specs=[_rows(_H), _rows(_H), _cols(_H), _cols(_H)],
        out_shape=[jax.ShapeDtypeStruct((_NP, _H), jnp.float32),
                   jax.ShapeDtypeStruct((_NP, _H), jnp.float32),
                   jax.ShapeDtypeStruct((_H, _NP), jnp.float32),
                   jax.ShapeDtypeStruct((_H, _NP), jnp.float32)],
    )(xh2, h, zrur, t3, Wc_all, bc2)


def _tcd_body(zcr_ref, t5_ref, u_ref, h_ref, hn_ref):
    c = jnp.tanh(zcr_ref[...] + _psum_t(t5_ref))
    u = u_ref[...]
    hn_ref[...] = u * h_ref[...] + (1.0 - u) * c


def _tcd(zcr, t5, u, h):
    return pl.pallas_call(
        _tcd_body,
        grid=(_NG,),
        in_specs=[_rows(_H), _parts(4, _H), _rows(_H), _rows(_H)],
        out_specs=_rows(_H),
        out_shape=jax.ShapeDtypeStruct((_NP, _H), jnp.float32),
    )(zcr, t5, u, h)


def kernel(x, edge_index, edge_weight, Wr_root, Wr_k, br, Wu_root, Wu_k, bu,
           Wc_root, Wc_k, bc, Wg, bg, Wd_root, Wd_k, bd, Wo, bo):
    combo = (edge_index[0] << 14) | edge_index[1]
    ew = edge_weight

    Wd_all = jnp.concatenate([Wd_root, Wd_k[0]], axis=1)
    Wru_all = jnp.concatenate(
        [Wr_root, Wu_root, Wr_k[0], Wu_k[0], Wr_k[1], Wu_k[1]], axis=1)
    Wc_all = jnp.concatenate([Wc_root, Wc_k[0], Wc_k[1]], axis=1)
    bru2 = jnp.concatenate([br, bu])[None, :]
    bg2 = bg[None, :]
    bd2 = bd[None, :]
    bo2 = bo[None, :]
    bc2 = bc[None, :]

    T = x.shape[1]
    xp = jnp.pad(x, ((0, 0), (0, 0), (0, _NP - _N), (0, 0)))
    h = jnp.zeros((_NP, _H), jnp.float32)
    gens, preds, reprs, states = [], [], [], []
    for t in range(T):
        x_s = xp[0, t]
        xh1, zdr, zd1T = _tca(x_s, h, Wg, bg2, Wd_all, bd2)
        p1 = _prop(32, zd1T, combo, ew)
        rep, xh2, zrur, zru1T, zru2T = _tcb(x_s, h, zdr, p1,
                                            Wo, bo2, Wru_all, bru2)
        t2 = _prop(64, zru2T, combo, ew)
        t3 = _prop(64, zru1T, t2, combo, ew)
        u, zcr, zc1T, zc2T = _tcc(xh2, h, zrur, t3, Wc_all, bc2)
        t4 = _prop(32, zc2T, combo, ew)
        t5 = _prop(32, zc1T, t4, combo, ew)
        h = _tcd(zcr, t5, u, h)
        gens.append(xh2[:_N])
        preds.append(xh1[:_N])
        reprs.append(rep[:_N])
        states.append(h[:_N])

    generations = jnp.stack(gens, axis=0)[None]
    predictions = jnp.stack(preds, axis=0)[None]
    representations = jnp.stack(reprs, axis=0)[None]
    states_arr = jnp.stack(states, axis=0)[None, None]
    return (generations, predictions, representations, states_arr)
